# baseline (speedup 1.0000x reference)
import functools

import jax
import jax.numpy as jnp
from jax import lax
from jax.experimental import pallas as pl
from jax.experimental.pallas import tpu as pltpu
from jax.experimental.pallas import tpu_sc as plsc

DIM = 64
MAXF = 432000
B_TOTAL = 1024 * 200
L = 16
NC, NS = 2, 16
NW = NC * NS
SUB = 128
ROWS_PER_W = B_TOTAL // NW
NSUB = ROWS_PER_W // SUB

_INV60 = 1.0 / 60.0

_mesh = plsc.VectorSubcoreMesh(core_axis_name="c", subcore_axis_name="s")


@functools.partial(
    pl.kernel,
    mesh=_mesh,
    compiler_params=pltpu.CompilerParams(use_tc_tiling_on_sc=True),
    out_type=jax.ShapeDtypeStruct((B_TOTAL // 2, 2 * DIM), jnp.float32),
    scratch_types=[
        pltpu.VMEM((ROWS_PER_W,), jnp.int32),
        pltpu.VMEM((ROWS_PER_W,), jnp.int32),
        pltpu.VMEM((ROWS_PER_W,), jnp.int32),
        pltpu.VMEM((2, SUB, 2 * DIM), jnp.float32),
        pltpu.VMEM((2, SUB, 2 * DIM), jnp.float32),
        pltpu.VMEM((2, SUB, 2 * DIM), jnp.float32),
        pltpu.SemaphoreType.DMA,
        pltpu.SemaphoreType.DMA,
    ],
)
def _encode(idx_hbm, ftab, petab, stab, out_hbm,
            idx_v, pidx_v, sidx_v, fbuf, pbuf, sbuf, sem0, sem1):
    wid = lax.axis_index("s") * NC + lax.axis_index("c")
    sems = (sem0, sem1)

    pltpu.sync_copy(idx_hbm.at[pl.ds(wid * ROWS_PER_W, ROWS_PER_W)], idx_v)

    def derive(j, carry):
        s = pl.ds(j * L, L)
        v = idx_v[s]
        pidx_v[s] = lax.shift_right_logical(v, 1)
        sidx_v[s] = (v.astype(jnp.float32) * _INV60).astype(jnp.int32)
        return carry

    lax.fori_loop(0, ROWS_PER_W // L, derive, 0)

    def gathers(j, b):
        isl = pl.ds(j * SUB, SUB)
        return [
            pltpu.make_async_copy(ftab.at[pidx_v.at[isl]], fbuf.at[b], sems[b]),
            pltpu.make_async_copy(petab.at[pidx_v.at[isl]], pbuf.at[b], sems[b]),
            pltpu.make_async_copy(stab.at[sidx_v.at[isl]], sbuf.at[b], sems[b]),
        ]

    def fire(j, b):
        for cp in gathers(j, b):
            cp.start()

    fire(0, 0)

    def pair(i2, carry):
        j0 = 2 * i2
        for b in range(2):
            j = j0 + b

            @pl.when(j < NSUB - 1)
            def _():
                fire(j + 1, 1 - b)

            for cp in gathers(j, b):
                cp.wait()

            def comb_row(r, c2):
                base = j * SUB + lax.bitwise_and(r, ~(L - 1))
                lane = lax.bitwise_and(r, L - 1)
                ivec = idx_v[pl.ds(pl.multiple_of(base, L), L)]
                iv = ivec.at[jnp.full((L,), lane, jnp.int32)].get(
                    mode="promise_in_bounds")
                hf = lax.bitwise_and(iv, 1).astype(jnp.float32)
                gf = 1.0 - hf
                half = lax.bitwise_and(r, 1) * DIM
                tr = lax.shift_right_logical(r, 1)
                for q in range(DIM // L):
                    lo = pl.ds(q * L, L)
                    hi = pl.ds(DIM + q * L, L)
                    c_lo = fbuf[b, r, lo] + pbuf[b, r, lo]
                    c_hi = fbuf[b, r, hi] + pbuf[b, r, hi]
                    res = (gf * c_lo + hf * c_hi
                           + sbuf[b, r, lo] + sbuf[b, r, hi])
                    fbuf[b, tr, pl.ds(half + q * L, L)] = res
                return c2

            lax.fori_loop(0, SUB, comb_row, 0)
            pltpu.sync_copy(
                fbuf.at[b, pl.ds(0, SUB // 2)],
                out_hbm.at[pl.ds(
                    pl.multiple_of((wid * ROWS_PER_W + j * SUB) // 2, SUB // 2),
                    SUB // 2)])
        return carry

    lax.fori_loop(0, NSUB // 2, pair, 0)


def kernel(frame_indices, frame_table, second_table, minute_table, pe):
    bsz, seq = frame_indices.shape
    idx = frame_indices.astype(jnp.int32).reshape(-1)
    fview = frame_table.reshape(MAXF // 2, 2 * DIM)
    peview = pe.reshape(MAXF // 2, 2 * DIM)
    small = jnp.concatenate(
        [second_table, jnp.repeat(minute_table, 60, axis=0)], axis=1)
    out = _encode(idx, fview, peview, small)
    return out.reshape(bsz, seq, DIM)
